# baseline (speedup 1.0000x reference)
import functools

import jax
import jax.numpy as jnp
from jax import lax
from jax.experimental import pallas as pl
from jax.experimental.pallas import tpu as pltpu
from jax.experimental.pallas import tpu_sc as plsc

B = 16384
L = 200
D = 64
NUM_WORKERS = 32
ROWS_PER_WORKER = B // NUM_WORKERS
STAGE_ROWS = 64
NUM_STAGES = ROWS_PER_WORKER // STAGE_ROWS


def _sc_partial_sums(tag_ids, table):
    mesh = plsc.VectorSubcoreMesh(core_axis_name="c", subcore_axis_name="s")

    @functools.partial(
        pl.kernel,
        out_type=jax.ShapeDtypeStruct((NUM_WORKERS, L, D), jnp.float32),
        mesh=mesh,
        compiler_params=pltpu.CompilerParams(use_tc_tiling_on_sc=False),
        scratch_types=[
            pltpu.VMEM((2, STAGE_ROWS, L), jnp.int32),
            pltpu.VMEM((2, L, D), jnp.float32),
            pltpu.VMEM((L, D), jnp.float32),
            pltpu.SemaphoreType.DMA,
            pltpu.SemaphoreType.DMA,
        ],
    )
    def sc_kernel(ids_hbm, table_hbm, out_hbm, ids_v, rows_v, acc_v,
                  sem_ids, sem_g):
        wid = lax.axis_index("s") * 2 + lax.axis_index("c")
        row0 = wid * ROWS_PER_WORKER

        zeros = jnp.zeros((16,), jnp.float32)

        @pl.loop(0, L)
        def _zero(j):
            for d in range(D // 16):
                acc_v[j, pl.ds(d * 16, 16)] = zeros

        def issue_gather(sbuf, r_local, rbuf):
            pltpu.async_copy(
                table_hbm.at[ids_v.at[sbuf, r_local, pl.ds(0, 128)]],
                rows_v.at[rbuf, pl.ds(0, 128), :], sem_g)
            pltpu.async_copy(
                table_hbm.at[ids_v.at[sbuf, r_local, pl.ds(128, L - 128)]],
                rows_v.at[rbuf, pl.ds(128, L - 128), :], sem_g)

        def wait_row(rbuf):
            pltpu.make_async_copy(
                table_hbm.at[pl.ds(0, L), :], rows_v.at[rbuf], sem_g).wait()

        pltpu.async_copy(
            ids_hbm.at[pl.ds(row0, STAGE_ROWS), :], ids_v.at[0],
            sem_ids).wait()
        issue_gather(0, 0, 0)

        @pl.loop(0, ROWS_PER_WORKER)
        def _row(r):
            nxt = r + 1
            s_pref = r // STAGE_ROWS + 1

            @pl.when(jnp.logical_and(r % STAGE_ROWS == 0,
                                     s_pref < NUM_STAGES))
            def _():
                pltpu.async_copy(
                    ids_hbm.at[pl.ds(row0 + s_pref * STAGE_ROWS,
                                     STAGE_ROWS), :],
                    ids_v.at[s_pref % 2], sem_ids)

            @pl.when(jnp.logical_and(nxt % STAGE_ROWS == 0,
                                     nxt < ROWS_PER_WORKER))
            def _():
                pltpu.make_async_copy(
                    ids_hbm.at[pl.ds(0, STAGE_ROWS), :],
                    ids_v.at[(nxt // STAGE_ROWS) % 2], sem_ids).wait()

            @pl.when(nxt < ROWS_PER_WORKER)
            def _():
                issue_gather((nxt // STAGE_ROWS) % 2, nxt % STAGE_ROWS,
                             nxt % 2)

            wait_row(r % 2)
            rbuf = r % 2

            @pl.loop(0, L, unroll=8)
            def _accum(j):
                for d in range(D // 16):
                    v = rows_v[rbuf, j, pl.ds(d * 16, 16)]
                    plsc.addupdate(acc_v.at[j, pl.ds(d * 16, 16)], v)

        pltpu.sync_copy(acc_v, out_hbm.at[wid])

    return sc_kernel(tag_ids, table)


def _mlp(partials, W1, b1, W2, b2, W3, b3):
    def body(p_ref, w1_ref, b1_ref, w2_ref, b2_ref, w3_ref, b3_ref, o_ref):
        x = jnp.sum(p_ref[...], axis=0) * (1.0 / B)
        x = jnp.maximum(
            jnp.dot(x, w1_ref[...], preferred_element_type=jnp.float32)
            + b1_ref[...], 0.0)
        x = jnp.maximum(
            jnp.dot(x, w2_ref[...], preferred_element_type=jnp.float32)
            + b2_ref[...], 0.0)
        o_ref[...] = (
            jnp.dot(x, w3_ref[...], preferred_element_type=jnp.float32)
            + b3_ref[...])

    return pl.pallas_call(
        body,
        out_shape=jax.ShapeDtypeStruct((L, D), jnp.float32),
    )(partials, W1, b1.reshape(1, D), W2, b2.reshape(1, D),
      W3, b3.reshape(1, D))


def kernel(tag_ids, table, W1, b1, W2, b2, W3, b3):
    tag_ids = tag_ids.astype(jnp.int32)
    partials = _sc_partial_sums(tag_ids, table)
    return _mlp(partials, W1, b1, W2, b2, W3, b3)


# speedup vs baseline: 1.9201x; 1.9201x over previous
import functools

import jax
import jax.numpy as jnp
from jax import lax
from jax.experimental import pallas as pl
from jax.experimental.pallas import tpu as pltpu
from jax.experimental.pallas import tpu_sc as plsc

B = 16384
L = 200
D = 64
NUM_WORKERS = 32
ROWS_PER_WORKER = B // NUM_WORKERS
STAGE_ROWS = 64
NUM_STAGES = ROWS_PER_WORKER // STAGE_ROWS


def _sc_partial_sums(tag_ids, table):
    mesh = plsc.VectorSubcoreMesh(core_axis_name="c", subcore_axis_name="s")

    @functools.partial(
        pl.kernel,
        out_type=jax.ShapeDtypeStruct((NUM_WORKERS, L, D), jnp.float32),
        mesh=mesh,
        compiler_params=pltpu.CompilerParams(use_tc_tiling_on_sc=False),
        scratch_types=[
            pltpu.VMEM((2, STAGE_ROWS, L), jnp.int32),
            pltpu.VMEM((2, L, D), jnp.float32),
            pltpu.VMEM((L, D), jnp.float32),
            pltpu.SemaphoreType.DMA,
            pltpu.SemaphoreType.DMA,
        ],
    )
    def sc_kernel(ids_hbm, table_hbm, out_hbm, ids_v, rows_v, acc_v,
                  sem_ids, sem_g):
        wid = lax.axis_index("s") * 2 + lax.axis_index("c")
        row0 = wid * ROWS_PER_WORKER

        zeros = jnp.zeros((16,), jnp.float32)

        @pl.loop(0, L)
        def _zero(j):
            for d in range(D // 16):
                acc_v[j, pl.ds(d * 16, 16)] = zeros

        def issue_gather(sbuf, r_local, rbuf):
            pltpu.async_copy(
                table_hbm.at[ids_v.at[sbuf, r_local, pl.ds(0, 128)]],
                rows_v.at[rbuf, pl.ds(0, 128), :], sem_g)
            pltpu.async_copy(
                table_hbm.at[ids_v.at[sbuf, r_local, pl.ds(128, L - 128)]],
                rows_v.at[rbuf, pl.ds(128, L - 128), :], sem_g)

        def wait_row(rbuf):
            pltpu.make_async_copy(
                table_hbm.at[pl.ds(0, L), :], rows_v.at[rbuf], sem_g).wait()

        def accumulate(rbuf):
            @pl.loop(0, L, unroll=8)
            def _accum(j):
                for d in range(D // 16):
                    v = rows_v[rbuf, j, pl.ds(d * 16, 16)]
                    plsc.addupdate(acc_v.at[j, pl.ds(d * 16, 16)], v)

        pltpu.async_copy(
            ids_hbm.at[pl.ds(row0, STAGE_ROWS), :], ids_v.at[0],
            sem_ids).wait()
        issue_gather(0, 0, 0)

        for s in range(NUM_STAGES):
            sb = s % 2
            if s + 1 < NUM_STAGES:
                pltpu.async_copy(
                    ids_hbm.at[pl.ds(row0 + (s + 1) * STAGE_ROWS,
                                     STAGE_ROWS), :],
                    ids_v.at[(s + 1) % 2], sem_ids)

            @pl.loop(0, STAGE_ROWS // 2 - 1)
            def _pair(p):
                issue_gather(sb, 2 * p + 1, 1)
                wait_row(0)
                accumulate(0)
                issue_gather(sb, 2 * p + 2, 0)
                wait_row(1)
                accumulate(1)

            issue_gather(sb, STAGE_ROWS - 1, 1)
            wait_row(0)
            accumulate(0)
            if s + 1 < NUM_STAGES:
                pltpu.make_async_copy(
                    ids_hbm.at[pl.ds(0, STAGE_ROWS), :],
                    ids_v.at[(s + 1) % 2], sem_ids).wait()
                issue_gather((s + 1) % 2, 0, 0)
            wait_row(1)
            accumulate(1)

        pltpu.sync_copy(acc_v, out_hbm.at[wid])

    return sc_kernel(tag_ids, table)


def _mlp(partials, W1, b1, W2, b2, W3, b3):
    def body(p_ref, w1_ref, b1_ref, w2_ref, b2_ref, w3_ref, b3_ref, o_ref):
        x = jnp.sum(p_ref[...], axis=0) * (1.0 / B)
        x = jnp.maximum(
            jnp.dot(x, w1_ref[...], preferred_element_type=jnp.float32)
            + b1_ref[...], 0.0)
        x = jnp.maximum(
            jnp.dot(x, w2_ref[...], preferred_element_type=jnp.float32)
            + b2_ref[...], 0.0)
        o_ref[...] = (
            jnp.dot(x, w3_ref[...], preferred_element_type=jnp.float32)
            + b3_ref[...])

    return pl.pallas_call(
        body,
        out_shape=jax.ShapeDtypeStruct((L, D), jnp.float32),
    )(partials, W1, b1.reshape(1, D), W2, b2.reshape(1, D),
      W3, b3.reshape(1, D))


def kernel(tag_ids, table, W1, b1, W2, b2, W3, b3):
    tag_ids = tag_ids.astype(jnp.int32)
    partials = _sc_partial_sums(tag_ids, table)
    return _mlp(partials, W1, b1, W2, b2, W3, b3)


# speedup vs baseline: 3.4995x; 1.8225x over previous
import functools

import jax
import jax.numpy as jnp
from jax import lax
from jax.experimental import pallas as pl
from jax.experimental.pallas import tpu as pltpu
from jax.experimental.pallas import tpu_sc as plsc

NUM_TAGS = 1000000
B = 16384
L = 200
D = 64
NUM_WORKERS = 32
ROWS_PER_WORKER = B // NUM_WORKERS
STAGE_ROWS = 64
NUM_STAGES = ROWS_PER_WORKER // STAGE_ROWS


def _sc_partial_sums(tag_ids, table):
    mesh = plsc.VectorSubcoreMesh(core_axis_name="c", subcore_axis_name="s")

    @functools.partial(
        pl.kernel,
        out_type=jax.ShapeDtypeStruct((NUM_WORKERS, L, D), jnp.float32),
        mesh=mesh,
        compiler_params=pltpu.CompilerParams(use_tc_tiling_on_sc=False),
        scratch_types=[
            pltpu.VMEM((2, STAGE_ROWS, L), jnp.int32),
            pltpu.VMEM((4, L, D), jnp.float32),
            pltpu.VMEM((L, D), jnp.float32),
            pltpu.SemaphoreType.DMA,
            pltpu.SemaphoreType.DMA,
        ],
    )
    def sc_kernel(ids_hbm, table_hbm, out_hbm, ids_v, rows_v, acc_v,
                  sem_ids, sem_g):
        wid = lax.axis_index("s") * 2 + lax.axis_index("c")
        row0 = wid * ROWS_PER_WORKER

        zeros = jnp.zeros((16,), jnp.float32)

        @pl.loop(0, L)
        def _zero(j):
            for d in range(D // 16):
                acc_v[j, pl.ds(d * 16, 16)] = zeros

        def issue_gather(sbuf, r_local, rbuf):
            pltpu.async_copy(
                table_hbm.at[ids_v.at[sbuf, r_local, pl.ds(0, 128)]],
                rows_v.at[rbuf, pl.ds(0, 128), :], sem_g)
            pltpu.async_copy(
                table_hbm.at[ids_v.at[sbuf, r_local, pl.ds(128, L - 128)]],
                rows_v.at[rbuf, pl.ds(128, L - 128), :], sem_g)

        def wait_row(rbuf):
            pltpu.make_async_copy(
                table_hbm.at[pl.ds(0, L), :], rows_v.at[rbuf], sem_g).wait()

        def acc_pair(ra, rb):
            @pl.loop(0, L, unroll=8)
            def _accum(j):
                for d in range(D // 16):
                    v = (rows_v[ra, j, pl.ds(d * 16, 16)]
                         + rows_v[rb, j, pl.ds(d * 16, 16)])
                    plsc.addupdate(acc_v.at[j, pl.ds(d * 16, 16)], v)

        pltpu.async_copy(
            ids_hbm.at[pl.ds(row0, STAGE_ROWS), :], ids_v.at[0],
            sem_ids).wait()
        issue_gather(0, 0, 0)
        issue_gather(0, 1, 1)

        for s in range(NUM_STAGES):
            sb = s % 2
            if s + 1 < NUM_STAGES:
                pltpu.async_copy(
                    ids_hbm.at[pl.ds(row0 + (s + 1) * STAGE_ROWS,
                                     STAGE_ROWS), :],
                    ids_v.at[(s + 1) % 2], sem_ids)

            @pl.loop(0, STAGE_ROWS // 4 - 1)
            def _quad(p):
                issue_gather(sb, 4 * p + 2, 2)
                issue_gather(sb, 4 * p + 3, 3)
                wait_row(0)
                wait_row(1)
                acc_pair(0, 1)
                issue_gather(sb, 4 * p + 4, 0)
                issue_gather(sb, 4 * p + 5, 1)
                wait_row(2)
                wait_row(3)
                acc_pair(2, 3)

            issue_gather(sb, STAGE_ROWS - 2, 2)
            issue_gather(sb, STAGE_ROWS - 1, 3)
            wait_row(0)
            wait_row(1)
            acc_pair(0, 1)
            if s + 1 < NUM_STAGES:
                pltpu.make_async_copy(
                    ids_hbm.at[pl.ds(0, STAGE_ROWS), :],
                    ids_v.at[(s + 1) % 2], sem_ids).wait()
                issue_gather((s + 1) % 2, 0, 0)
                issue_gather((s + 1) % 2, 1, 1)
            wait_row(2)
            wait_row(3)
            acc_pair(2, 3)

        pltpu.sync_copy(acc_v, out_hbm.at[wid])

    return sc_kernel(tag_ids, table)


def _mlp(partials, W1, b1, W2, b2, W3, b3):
    def body(p_ref, w1_ref, b1_ref, w2_ref, b2_ref, w3_ref, b3_ref, o_ref):
        x = jnp.sum(p_ref[...], axis=0) * (1.0 / B)
        x = jnp.maximum(
            jnp.dot(x, w1_ref[...], preferred_element_type=jnp.float32)
            + b1_ref[...], 0.0)
        x = jnp.maximum(
            jnp.dot(x, w2_ref[...], preferred_element_type=jnp.float32)
            + b2_ref[...], 0.0)
        o_ref[...] = (
            jnp.dot(x, w3_ref[...], preferred_element_type=jnp.float32)
            + b3_ref[...])

    return pl.pallas_call(
        body,
        out_shape=jax.ShapeDtypeStruct((L, D), jnp.float32),
    )(partials, W1, b1.reshape(1, D), W2, b2.reshape(1, D),
      W3, b3.reshape(1, D))


_TW = 16384
_TG = (NUM_TAGS + _TW - 1) // _TW
_TROWS = _TG * (_TW // 2)


_TAIL = NUM_TAGS - (_TG - 1) * _TW


def _repack_table(table):
    tableT = table.T
    tail2 = tableT[:, (_TG - 1) * _TW:]

    def body(x_ref, tail_ref, o_ref):
        i = pl.program_id(0)

        @pl.when(i < _TG - 1)
        def _():
            x = x_ref[...]
            x2 = jnp.concatenate(
                [x[:, :_TW // 2], x[:, _TW // 2:]], axis=0)
            o_ref[...] = x2.T

        @pl.when(i == _TG - 1)
        def _():
            xt = tail_ref[...]
            x2 = jnp.concatenate(
                [xt[:, :_TAIL // 2], xt[:, _TAIL // 2:]], axis=0)
            o_ref[pl.ds(0, _TAIL // 2), :] = x2.T

    out = pl.pallas_call(
        body,
        grid=(_TG,),
        in_specs=[
            pl.BlockSpec((D, _TW), lambda i: (0, jnp.minimum(i, _TG - 2))),
            pl.BlockSpec((D, _TAIL), lambda i: (0, 0)),
        ],
        out_specs=pl.BlockSpec((_TW // 2, 128), lambda i: (i, 0)),
        out_shape=jax.ShapeDtypeStruct((_TROWS, 128), jnp.float32),
    )(tableT, tail2)
    return jnp.reshape(out, (_TROWS * 2, D))


def kernel(tag_ids, table, W1, b1, W2, b2, W3, b3):
    tag_ids = tag_ids.astype(jnp.int32)
    table_lin = _repack_table(table)
    j = tag_ids & (_TW - 1)
    f_main = (tag_ids - j) + ((j & (_TW // 2 - 1)) << 1) + (j >> 13)
    base = (_TG - 1) * _TW
    jt = tag_ids - base
    f_tail = base + ((jt % (_TAIL // 2)) << 1) + (jt // (_TAIL // 2))
    ids_f = jnp.where(tag_ids < base, f_main, f_tail)
    ids_lin = lax.optimization_barrier(
        jnp.reshape(ids_f, (B * L // 128, 128)))
    ids_lin = jnp.reshape(ids_lin, (B, L))
    partials = _sc_partial_sums(ids_lin, table_lin)
    return _mlp(partials, W1, b1, W2, b2, W3, b3)
